# Initial kernel scaffold; baseline (speedup 1.0000x reference)
import jax
import jax.numpy as jnp
from jax.experimental import pallas as pl


def kernel(x, queue_x):
    raise NotImplementedError("write your pallas kernel here")


# speedup vs baseline: 4.2661x; 4.2661x over previous
import functools

import jax
import jax.numpy as jnp
from jax import lax
from jax.experimental import pallas as pl
from jax.experimental.pallas import tpu as pltpu
from jax.experimental.pallas import tpu_sc as plsc

_QB = 400


def _argmax_body(nq, qb, q_ref, xt_ref, idx_ref, rmax_ref, ridx_ref):
    i = pl.program_id(0)
    sim = lax.dot_general(
        q_ref[...], xt_ref[...],
        dimension_numbers=(((1,), (0,)), ((), ())),
        preferred_element_type=jnp.float32,
    )
    riota = lax.broadcasted_iota(jnp.int32, sim.shape, 0)
    bm = jnp.max(sim, axis=0)
    cand = jnp.where(sim < bm[None, :], jnp.int32(2**30), riota)
    bi = jnp.min(cand, axis=0)

    @pl.when(i == 0)
    def _():
        rmax_ref[...] = bm
        ridx_ref[...] = bi

    @pl.when(i > 0)
    def _():
        prev_m = rmax_ref[...]
        prev_i = ridx_ref[...]
        better = bm > prev_m
        rmax_ref[...] = jnp.where(better, bm, prev_m)
        ridx_ref[...] = jnp.where(better, bi + i * qb, prev_i)

    @pl.when(i == nq - 1)
    def _():
        idx_ref[...] = ridx_ref[...]


def _nn_argmax(x, queue_x, qb=_QB):
    b, d = x.shape
    n = queue_x.shape[0]
    nq = n // qb
    xt = x.T
    return pl.pallas_call(
        functools.partial(_argmax_body, nq, qb),
        grid=(nq,),
        in_specs=[
            pl.BlockSpec((qb, d), lambda i: (i, 0)),
            pl.BlockSpec((d, b), lambda i: (0, 0)),
        ],
        out_specs=pl.BlockSpec((b,), lambda i: (0,)),
        out_shape=jax.ShapeDtypeStruct((b,), jnp.int32),
        scratch_shapes=[
            pltpu.VMEM((b,), jnp.float32),
            pltpu.VMEM((b,), jnp.int32),
        ],
    )(queue_x, xt)


def _gather_rows(queue_x, idx):
    n, d = queue_x.shape
    b = idx.shape[0]
    info = plsc.get_sparse_core_info()
    nw = info.num_cores * info.num_subcores
    bpw = b // nw
    mesh = plsc.VectorSubcoreMesh(core_axis_name="c", subcore_axis_name="s")

    @functools.partial(
        pl.kernel,
        mesh=mesh,
        out_type=jax.ShapeDtypeStruct((b, d), jnp.float32),
        scratch_types=[
            pltpu.VMEM((bpw,), jnp.int32),
            pltpu.VMEM((bpw, d), jnp.float32),
            pltpu.SemaphoreType.DMA,
        ],
    )
    def gk(table_hbm, idx_hbm, out_hbm, idx_v, rows_v, sem):
        wid = lax.axis_index("s") * info.num_cores + lax.axis_index("c")
        base = wid * bpw
        pltpu.sync_copy(idx_hbm.at[pl.ds(base, bpw)], idx_v)
        pltpu.async_copy(table_hbm.at[idx_v], rows_v, sem).wait()
        pltpu.sync_copy(rows_v, out_hbm.at[pl.ds(base, bpw)])

    return gk(queue_x, idx)


def kernel(x, queue_x):
    idx = _nn_argmax(x, queue_x)
    return _gather_rows(queue_x, idx)
